# baseline (speedup 1.0000x reference)
import jax
import jax.numpy as jnp
from jax import lax
from jax.experimental import pallas as pl
from jax.experimental.pallas import tpu as pltpu
from jax.experimental.pallas import tpu_sc as plsc

VOCAB = 1_000_000
D = 32
B = 16384
H = 200
HID = 128
OUT = 64

NC, NS = 2, 16
NW = NC * NS
ROWS_PER_W = B // NW
CHUNK_ROWS = 8
CHUNK_IDX = CHUNK_ROWS * H
N_CHUNKS = ROWS_PER_W // CHUNK_ROWS

ROW_SLICES = [(0, 128), (128, 72)]
OUT_BATCH = 8


def _pool_body(x_hbm, tab_hbm, out_hbm,
               i0, i1, r0, r1, out_v, isem0, isem1, gsem0, gsem1):
    c = lax.axis_index("c")
    s = lax.axis_index("s")
    wid = s * NC + c
    chunk0 = wid * N_CHUNKS

    zeros = jnp.zeros((16,), jnp.float32)

    def idx_copy(g, idx_v, isem):
        return pltpu.make_async_copy(
            x_hbm.at[pl.ds((chunk0 + g) * CHUNK_ROWS, CHUNK_ROWS), :],
            idx_v, isem)

    def gathers(idx_v, rows_v, gsem):
        return [pltpu.make_async_copy(
                    tab_hbm.at[idx_v.at[r, pl.ds(off, sz)]],
                    rows_v.at[pl.ds(r * H + off, sz)], gsem)
                for r in range(CHUNK_ROWS)
                for off, sz in ROW_SLICES]

    def accumulate(g, rows_v):
        vbase = (g % OUT_BATCH) * CHUNK_ROWS
        for r in range(CHUNK_ROWS):
            rb = r * H

            def acc_body(j, ac):
                w = plsc.bitcast(rows_v[rb + j, :], jnp.int32)
                even = plsc.bitcast(lax.shift_left(w, 16), jnp.float32)
                odd = plsc.bitcast(w & jnp.int32(-65536), jnp.float32)
                return ac[0] + even, ac[1] + odd

            a0, a1 = lax.fori_loop(0, H, acc_body, (zeros, zeros), unroll=8)
            out_v[vbase + r, pl.ds(0, 16)] = a0
            out_v[vbase + r, pl.ds(16, 16)] = a1

        @pl.when(g % OUT_BATCH == OUT_BATCH - 1)
        def _():
            pltpu.sync_copy(
                out_v,
                out_hbm.at[pl.ds(
                    (chunk0 + g - (OUT_BATCH - 1)) * CHUNK_ROWS,
                    OUT_BATCH * CHUNK_ROWS)])

    idx_copy(0, i0, isem0).start()
    idx_copy(0, i0, isem0).wait()
    for cp in gathers(i0, r0, gsem0):
        cp.start()
    idx_copy(1, i1, isem1).start()

    def pair_body(p, carry):
        g = p * 2
        idx_copy(g + 1, i1, isem1).wait()
        for cp in gathers(i1, r1, gsem1):
            cp.start()
        for cp in gathers(i0, r0, gsem0):
            cp.wait()

        @pl.when(p < N_CHUNKS // 2 - 1)
        def _():
            idx_copy(g + 2, i0, isem0).start()

        accumulate(g, r0)

        @pl.when(p < N_CHUNKS // 2 - 1)
        def _():
            idx_copy(g + 2, i0, isem0).wait()
            for cp in gathers(i0, r0, gsem0):
                cp.start()

        for cp in gathers(i1, r1, gsem1):
            cp.wait()

        @pl.when(p < N_CHUNKS // 2 - 1)
        def _():
            idx_copy(g + 3, i1, isem1).start()

        accumulate(g + 1, r1)
        return carry

    lax.fori_loop(0, N_CHUNKS // 2, pair_body, 0)


_pool = pl.kernel(
    _pool_body,
    mesh=plsc.VectorSubcoreMesh(core_axis_name="c", subcore_axis_name="s"),
    out_type=jax.ShapeDtypeStruct((B, D), jnp.float32),
    scratch_types=[
        pltpu.VMEM((CHUNK_ROWS, H), jnp.int32),
        pltpu.VMEM((CHUNK_ROWS, H), jnp.int32),
        pltpu.VMEM((CHUNK_IDX, D), jnp.bfloat16),
        pltpu.VMEM((CHUNK_IDX, D), jnp.bfloat16),
        pltpu.VMEM((OUT_BATCH * CHUNK_ROWS, D), jnp.float32),
        pltpu.SemaphoreType.DMA,
        pltpu.SemaphoreType.DMA,
        pltpu.SemaphoreType.DMA,
        pltpu.SemaphoreType.DMA,
    ],
    compiler_params=pltpu.CompilerParams(use_tc_tiling_on_sc=False,
                                         needs_layout_passes=False),
)


def _mlp_body(p_ref, w1_ref, b1_ref, w2_ref, b2_ref, o_ref):
    p = p_ref[...] * (1.0 / H)
    h = jnp.maximum(
        jnp.dot(p, w1_ref[...], preferred_element_type=jnp.float32)
        + b1_ref[...], 0.0)
    o_ref[...] = (jnp.dot(h, w2_ref[...], preferred_element_type=jnp.float32)
                  + b2_ref[...])


MB = 2048

_mlp = pl.pallas_call(
    _mlp_body,
    grid=(B // MB,),
    in_specs=[
        pl.BlockSpec((MB, D), lambda i: (i, 0)),
        pl.BlockSpec((D, HID), lambda i: (0, 0)),
        pl.BlockSpec((1, HID), lambda i: (0, 0)),
        pl.BlockSpec((HID, OUT), lambda i: (0, 0)),
        pl.BlockSpec((1, OUT), lambda i: (0, 0)),
    ],
    out_specs=pl.BlockSpec((MB, OUT), lambda i: (i, 0)),
    out_shape=jax.ShapeDtypeStruct((B, OUT), jnp.float32),
)


def kernel(x, emb_table, W1, b1, W2, b2):
    tab_bf = emb_table.astype(jnp.bfloat16)
    pooled = _pool(x.astype(jnp.int32), tab_bf)
    w1p = jnp.concatenate([W1[0::2], W1[1::2]], axis=0)
    return _mlp(pooled, w1p, b1.reshape(1, HID), W2, b2.reshape(1, OUT))


# speedup vs baseline: 1.0504x; 1.0504x over previous
import jax
import jax.numpy as jnp
from jax import lax
from jax.experimental import pallas as pl
from jax.experimental.pallas import tpu as pltpu
from jax.experimental.pallas import tpu_sc as plsc

VOCAB = 1_000_000
D = 32
B = 16384
H = 200
HID = 128
OUT = 64

NC, NS = 2, 16
NW = NC * NS
ROWS_PER_W = B // NW
CHUNK_ROWS = 8
CHUNK_IDX = CHUNK_ROWS * H
N_CHUNKS = ROWS_PER_W // CHUNK_ROWS

ROW_SLICES = [(0, 128), (128, 72)]
OUT_BATCH = 8


def _pool_body(x_hbm, tab_hbm, out_hbm,
               i0, i1, r0, r1, out_v, isem0, isem1, gsem0, gsem1):
    c = lax.axis_index("c")
    s = lax.axis_index("s")
    wid = s * NC + c
    chunk0 = wid * N_CHUNKS

    zeros = jnp.zeros((16,), jnp.float32)

    def idx_copy(g, idx_v, isem):
        return pltpu.make_async_copy(
            x_hbm.at[pl.ds((chunk0 + g) * CHUNK_ROWS, CHUNK_ROWS), :],
            idx_v, isem)

    def gathers(idx_v, rows_v, gsem):
        return [pltpu.make_async_copy(
                    tab_hbm.at[idx_v.at[r, pl.ds(off, sz)]],
                    rows_v.at[pl.ds(r * H + off, sz)], gsem)
                for r in range(CHUNK_ROWS)
                for off, sz in ROW_SLICES]

    def accumulate(g, rows_v):
        vbase = (g % OUT_BATCH) * CHUNK_ROWS
        for r in range(CHUNK_ROWS):
            rb = r * H

            def acc_body(j, ac):
                row = rb + j
                return (ac[0] + rows_v[row, pl.ds(0, 16)],
                        ac[1] + rows_v[row, pl.ds(16, 16)])

            a0, a1 = lax.fori_loop(0, H, acc_body, (zeros, zeros), unroll=8)
            out_v[vbase + r, pl.ds(0, 16)] = a0
            out_v[vbase + r, pl.ds(16, 16)] = a1

        @pl.when(g % OUT_BATCH == OUT_BATCH - 1)
        def _():
            pltpu.sync_copy(
                out_v,
                out_hbm.at[pl.ds(
                    (chunk0 + g - (OUT_BATCH - 1)) * CHUNK_ROWS,
                    OUT_BATCH * CHUNK_ROWS)])

    idx_copy(0, i0, isem0).start()
    idx_copy(0, i0, isem0).wait()
    for cp in gathers(i0, r0, gsem0):
        cp.start()
    idx_copy(1, i1, isem1).start()

    def pair_body(p, carry):
        g = p * 2
        idx_copy(g + 1, i1, isem1).wait()
        for cp in gathers(i1, r1, gsem1):
            cp.start()
        for cp in gathers(i0, r0, gsem0):
            cp.wait()

        @pl.when(p < N_CHUNKS // 2 - 1)
        def _():
            idx_copy(g + 2, i0, isem0).start()

        accumulate(g, r0)

        @pl.when(p < N_CHUNKS // 2 - 1)
        def _():
            idx_copy(g + 2, i0, isem0).wait()
            for cp in gathers(i0, r0, gsem0):
                cp.start()

        for cp in gathers(i1, r1, gsem1):
            cp.wait()

        @pl.when(p < N_CHUNKS // 2 - 1)
        def _():
            idx_copy(g + 3, i1, isem1).start()

        accumulate(g + 1, r1)
        return carry

    lax.fori_loop(0, N_CHUNKS // 2, pair_body, 0)


_pool = pl.kernel(
    _pool_body,
    mesh=plsc.VectorSubcoreMesh(core_axis_name="c", subcore_axis_name="s"),
    out_type=jax.ShapeDtypeStruct((B, D), jnp.float32),
    scratch_types=[
        pltpu.VMEM((CHUNK_ROWS, H), jnp.int32),
        pltpu.VMEM((CHUNK_ROWS, H), jnp.int32),
        pltpu.VMEM((CHUNK_IDX, D), jnp.float32),
        pltpu.VMEM((CHUNK_IDX, D), jnp.float32),
        pltpu.VMEM((OUT_BATCH * CHUNK_ROWS, D), jnp.float32),
        pltpu.SemaphoreType.DMA,
        pltpu.SemaphoreType.DMA,
        pltpu.SemaphoreType.DMA,
        pltpu.SemaphoreType.DMA,
    ],
    compiler_params=pltpu.CompilerParams(use_tc_tiling_on_sc=False),
)


QUARTER = VOCAB // 4
DB = 2000


def _depad_body(t0, t1, t2, t3, o_ref):
    o_ref[:, 0:32] = t0[...]
    o_ref[:, 32:64] = t1[...]
    o_ref[:, 64:96] = t2[...]
    o_ref[:, 96:128] = t3[...]


_depad = pl.pallas_call(
    _depad_body,
    grid=(QUARTER // DB,),
    in_specs=[
        pl.BlockSpec((DB, D), lambda i, q=q: (QUARTER // DB * q + i, 0))
        for q in range(4)
    ],
    out_specs=pl.BlockSpec((DB, 128), lambda i: (i, 0)),
    out_shape=jax.ShapeDtypeStruct((QUARTER, 128), jnp.float32),
)


def _mlp_body(p_ref, w1_ref, b1_ref, w2_ref, b2_ref, o_ref):
    p = p_ref[...] * (1.0 / H)
    h = jnp.maximum(
        jnp.dot(p, w1_ref[...], preferred_element_type=jnp.float32)
        + b1_ref[...], 0.0)
    o_ref[...] = (jnp.dot(h, w2_ref[...], preferred_element_type=jnp.float32)
                  + b2_ref[...])


MB = 2048

_mlp = pl.pallas_call(
    _mlp_body,
    grid=(B // MB,),
    in_specs=[
        pl.BlockSpec((MB, D), lambda i: (i, 0)),
        pl.BlockSpec((D, HID), lambda i: (0, 0)),
        pl.BlockSpec((1, HID), lambda i: (0, 0)),
        pl.BlockSpec((HID, OUT), lambda i: (0, 0)),
        pl.BlockSpec((1, OUT), lambda i: (0, 0)),
    ],
    out_specs=pl.BlockSpec((MB, OUT), lambda i: (i, 0)),
    out_shape=jax.ShapeDtypeStruct((B, OUT), jnp.float32),
)


def kernel(x, emb_table, W1, b1, W2, b2):
    tab_lin = _depad(emb_table, emb_table, emb_table, emb_table)
    tab_lin = tab_lin.reshape(VOCAB, D)
    xi = x.astype(jnp.int32)
    xp = 4 * xi - (VOCAB - 1) * (xi // QUARTER)
    pooled = _pool(xp, tab_lin)
    return _mlp(pooled, W1, b1.reshape(1, HID), W2, b2.reshape(1, OUT))


# speedup vs baseline: 1.1193x; 1.0655x over previous
import jax
import jax.numpy as jnp
from jax import lax
from jax.experimental import pallas as pl
from jax.experimental.pallas import tpu as pltpu
from jax.experimental.pallas import tpu_sc as plsc

VOCAB = 1_000_000
D = 32
B = 16384
H = 200
HID = 128
OUT = 64

NC, NS = 2, 16
NW = NC * NS
ROWS_PER_W = B // NW
CHUNK_ROWS = 8
CHUNK_IDX = CHUNK_ROWS * H
N_CHUNKS = ROWS_PER_W // CHUNK_ROWS

ROW_SLICES = [(0, 128), (128, 72)]
OUT_BATCH = 8


def _pool_body(x_hbm, tab_hbm, out_hbm,
               i0, i1, r0, r1, out_v, isem0, isem1, gsem0, gsem1):
    c = lax.axis_index("c")
    s = lax.axis_index("s")
    wid = s * NC + c
    chunk0 = wid * N_CHUNKS

    zeros = jnp.zeros((16,), jnp.float32)

    def idx_copy(g, idx_v, isem):
        return pltpu.make_async_copy(
            x_hbm.at[pl.ds((chunk0 + g) * CHUNK_ROWS, CHUNK_ROWS), :],
            idx_v, isem)

    def gathers(idx_v, rows_v, gsem):
        return [pltpu.make_async_copy(
                    tab_hbm.at[idx_v.at[r, pl.ds(off, sz)]],
                    rows_v.at[pl.ds(r * H + off, sz)], gsem)
                for r in range(CHUNK_ROWS)
                for off, sz in ROW_SLICES]

    def accumulate(g, rows_v):
        vbase = (g % OUT_BATCH) * CHUNK_ROWS
        for r in range(CHUNK_ROWS):
            rb = r * H

            def acc_body(j, ac):
                row = rb + j
                return (ac[0] + rows_v[row, pl.ds(0, 16)],
                        ac[1] + rows_v[row, pl.ds(16, 16)])

            a0, a1 = lax.fori_loop(0, H, acc_body, (zeros, zeros), unroll=8)
            out_v[vbase + r, pl.ds(0, 16)] = a0
            out_v[vbase + r, pl.ds(16, 16)] = a1

        @pl.when(g % OUT_BATCH == OUT_BATCH - 1)
        def _():
            pltpu.sync_copy(
                out_v,
                out_hbm.at[pl.ds(
                    (chunk0 + g - (OUT_BATCH - 1)) * CHUNK_ROWS,
                    OUT_BATCH * CHUNK_ROWS)])

    idx_copy(0, i0, isem0).start()
    idx_copy(0, i0, isem0).wait()
    for cp in gathers(i0, r0, gsem0):
        cp.start()
    idx_copy(1, i1, isem1).start()

    def pair_body(p, carry):
        g = p * 2
        idx_copy(g + 1, i1, isem1).wait()
        for cp in gathers(i1, r1, gsem1):
            cp.start()
        for cp in gathers(i0, r0, gsem0):
            cp.wait()

        @pl.when(p < N_CHUNKS // 2 - 1)
        def _():
            idx_copy(g + 2, i0, isem0).start()

        accumulate(g, r0)

        @pl.when(p < N_CHUNKS // 2 - 1)
        def _():
            idx_copy(g + 2, i0, isem0).wait()
            for cp in gathers(i0, r0, gsem0):
                cp.start()

        for cp in gathers(i1, r1, gsem1):
            cp.wait()

        @pl.when(p < N_CHUNKS // 2 - 1)
        def _():
            idx_copy(g + 3, i1, isem1).start()

        accumulate(g + 1, r1)
        return carry

    lax.fori_loop(0, N_CHUNKS // 2, pair_body, 0)


_pool = pl.kernel(
    _pool_body,
    mesh=plsc.VectorSubcoreMesh(core_axis_name="c", subcore_axis_name="s"),
    out_type=jax.ShapeDtypeStruct((B, D), jnp.float32),
    scratch_types=[
        pltpu.VMEM((CHUNK_ROWS, H), jnp.int32),
        pltpu.VMEM((CHUNK_ROWS, H), jnp.int32),
        pltpu.VMEM((CHUNK_IDX, D), jnp.float32),
        pltpu.VMEM((CHUNK_IDX, D), jnp.float32),
        pltpu.VMEM((OUT_BATCH * CHUNK_ROWS, D), jnp.float32),
        pltpu.SemaphoreType.DMA,
        pltpu.SemaphoreType.DMA,
        pltpu.SemaphoreType.DMA,
        pltpu.SemaphoreType.DMA,
    ],
    compiler_params=pltpu.CompilerParams(use_tc_tiling_on_sc=False),
)


def _mlp_body(p_ref, w1_ref, b1_ref, w2_ref, b2_ref, o_ref):
    p = p_ref[...] * (1.0 / H)
    h = jnp.maximum(
        jnp.dot(p, w1_ref[...], preferred_element_type=jnp.float32)
        + b1_ref[...], 0.0)
    o_ref[...] = (jnp.dot(h, w2_ref[...], preferred_element_type=jnp.float32)
                  + b2_ref[...])


MB = 2048

_mlp = pl.pallas_call(
    _mlp_body,
    grid=(B // MB,),
    in_specs=[
        pl.BlockSpec((MB, D), lambda i: (i, 0)),
        pl.BlockSpec((D, HID), lambda i: (0, 0)),
        pl.BlockSpec((1, HID), lambda i: (0, 0)),
        pl.BlockSpec((HID, OUT), lambda i: (0, 0)),
        pl.BlockSpec((1, OUT), lambda i: (0, 0)),
    ],
    out_specs=pl.BlockSpec((MB, OUT), lambda i: (i, 0)),
    out_shape=jax.ShapeDtypeStruct((B, OUT), jnp.float32),
)


def kernel(x, emb_table, W1, b1, W2, b2):
    pooled = _pool(x.astype(jnp.int32), emb_table)
    return _mlp(pooled, W1, b1.reshape(1, HID), W2, b2.reshape(1, OUT))
